# baseline (speedup 1.0000x reference)
import functools

import jax
import jax.numpy as jnp
from jax import lax
from jax.experimental import pallas as pl
from jax.experimental.pallas import tpu as pltpu
from jax.experimental.pallas import tpu_sc as plsc

_B = 64
_IMG = 224
_P = 16
_D = 768
_K = 512
_H = _IMG // _P
_N = _B * _H * _H
_MASK_RATIO = 0.4

_BI = 2
_BM = _BI * _H * _H
_G = _B // _BI

_NC = 2
_NS = 16
_NW = _NC * _NS
_RPW = _N // _NW
_CH = 56
_NCH = _RPW // _CH


def _vq_block(p_ref, pw_ref, pb_ref, m_ref, cb_ref, cbn_ref, idx_ref, minv_ref):
    p6 = p_ref[...].reshape(_BI, 3, _H, _P, _H, _P)
    patches = jnp.transpose(p6, (0, 2, 4, 1, 3, 5)).reshape(_BM, _D)
    tok = lax.dot_general(patches, pw_ref[...], (((1,), (1,)), ((), ())),
                          preferred_element_type=jnp.float32)
    tok = tok + pb_ref[...]
    tok = jnp.where(m_ref[...] != 0.0, 0.0, tok)
    s = lax.dot_general(tok, cb_ref[...], (((1,), (1,)), ((), ())),
                        preferred_element_type=jnp.float32)
    r = jnp.sum(tok * tok, axis=1, keepdims=True)
    d = (r + cbn_ref[...]) - 2.0 * s
    minv = jnp.min(d, axis=1, keepdims=True)
    ks = lax.broadcasted_iota(jnp.int32, d.shape, 1)
    idx = jnp.min(jnp.where(d == minv, ks, _K), axis=1)
    idx_ref[...] = idx.astype(jnp.int32).reshape(_BM, 1)
    minv_ref[...] = minv.reshape(_BM, 1)


def _encode_vq(patches, pw, pb, maskf, cb, cbn):
    nimg = patches.shape[0]
    ntok = nimg * _H * _H
    return pl.pallas_call(
        _vq_block,
        grid=(nimg // _BI,),
        in_specs=[
            pl.BlockSpec((_BI, 3, _IMG, _IMG), lambda i: (i, 0, 0, 0)),
            pl.BlockSpec((_D, _D), lambda i: (0, 0)),
            pl.BlockSpec((1, _D), lambda i: (0, 0)),
            pl.BlockSpec((_BM, 1), lambda i: (i, 0)),
            pl.BlockSpec((_K, _D), lambda i: (0, 0)),
            pl.BlockSpec((1, _K), lambda i: (0, 0)),
        ],
        out_specs=[
            pl.BlockSpec((_BM, 1), lambda i: (i, 0)),
            pl.BlockSpec((_BM, 1), lambda i: (i, 0)),
        ],
        out_shape=[
            jax.ShapeDtypeStruct((ntok, 1), jnp.int32),
            jax.ShapeDtypeStruct((ntok, 1), jnp.float32),
        ],
    )(patches, pw, pb, maskf, cb, cbn)


def _dec_cb_block(cb_ref, dw_ref, db_ref, out_ref):
    out_ref[...] = lax.dot_general(
        cb_ref[...], dw_ref[...], (((1,), (1,)), ((), ())),
        preferred_element_type=jnp.float32) + db_ref[...]


def _decode_codebook(cb, dw, db):
    return pl.pallas_call(
        _dec_cb_block,
        out_shape=jax.ShapeDtypeStruct((_K, _D), jnp.float32),
    )(cb, dw, db)


def _sc_gather_rows_quarter(table, idx2):
    mesh = plsc.VectorSubcoreMesh(core_axis_name="c", subcore_axis_name="s")

    @functools.partial(
        pl.kernel,
        out_type=jax.ShapeDtypeStruct((_NW, 104, _D), jnp.float32),
        mesh=mesh,
        scratch_types=[
            pltpu.VMEM((104,), jnp.int32),
            pltpu.VMEM((56, _D), jnp.float32),
            pltpu.SemaphoreType.DMA,
        ],
    )
    def gk(table_hbm, idx_hbm, out_hbm, idx_v, rows_v, sem):
        wid = lax.axis_index("s") * _NC + lax.axis_index("c")
        pltpu.sync_copy(idx_hbm.at[wid], idx_v)
        for j, (off, sz) in enumerate(((0, 56), (56, 48))):
            src = table_hbm.at[idx_v.at[pl.ds(off, sz)]]
            dst = rows_v.at[pl.ds(0, sz)]
            pltpu.async_copy(src, dst, sem).wait()
            pltpu.sync_copy(dst, out_hbm.at[wid, pl.ds(off, sz)])

    return gk(table, idx2)


def kernel(x, patch_w, patch_b, codebook_w, dec_w, dec_b):
    pw = patch_w.reshape(_D, _D)
    pb = patch_b.reshape(1, _D)
    mask = jax.random.uniform(jax.random.key(42), (_B, _H * _H)) < _MASK_RATIO
    maskf = mask.astype(jnp.float32).reshape(_N, 1)
    cbn = jnp.sum(codebook_w ** 2, axis=1).reshape(1, _K)
    dec_cb = _decode_codebook(codebook_w, dec_w, dec_b.reshape(1, _D))

    hb = _B // 4
    ht = _N // 4
    recons, msums = [], []
    for h in range(4):
        idx_h, minv_h = _encode_vq(
            x[h * hb:(h + 1) * hb], pw, pb,
            maskf[h * ht:(h + 1) * ht], codebook_w, cbn)
        idx_p = jnp.pad(idx_h.reshape(_NW, 98), ((0, 0), (0, 6)))
        slab = _sc_gather_rows_quarter(dec_cb, idx_p)
        toks = slab[:, :98, :].reshape(hb, _H, _H, _P, _P, 3)
        recons.append(toks.transpose(0, 5, 1, 3, 2, 4)
                          .reshape(hb, 3, _IMG, _IMG))
        msums.append(jnp.sum(minv_h))
    recon = jnp.concatenate(recons, axis=0)
    m = (msums[0] + msums[1] + msums[2] + msums[3]) * (1.0 / (_N * _D))
    vq_loss = m + 0.25 * m
    return recon, vq_loss


# speedup vs baseline: 1.1724x; 1.1724x over previous
import functools

import jax
import jax.numpy as jnp
from jax import lax
from jax.experimental import pallas as pl
from jax.experimental.pallas import tpu as pltpu
from jax.experimental.pallas import tpu_sc as plsc

_B = 64
_IMG = 224
_P = 16
_D = 768
_K = 512
_H = _IMG // _P
_N = _B * _H * _H
_MASK_RATIO = 0.4

_BI = 2
_BM = _BI * _H * _H
_G = _B // _BI

_NC = 2
_NS = 16
_NW = _NC * _NS
_RPW = _N // _NW
_CH = 56
_NCH = _RPW // _CH


def _vq_block(p_ref, pw_ref, pb_ref, m_ref, cb_ref, cbn_ref, idx_ref, minv_ref):
    p6 = p_ref[...].reshape(_BI, 3, _H, _P, _H, _P)
    patches = jnp.transpose(p6, (0, 2, 4, 1, 3, 5)).reshape(_BM, _D)
    tok = lax.dot_general(patches, pw_ref[...], (((1,), (1,)), ((), ())),
                          preferred_element_type=jnp.float32)
    tok = tok + pb_ref[...]
    tok = jnp.where(m_ref[...] != 0.0, 0.0, tok)
    s = lax.dot_general(tok, cb_ref[...], (((1,), (1,)), ((), ())),
                        preferred_element_type=jnp.float32)
    r = jnp.sum(tok * tok, axis=1, keepdims=True)
    d = (r + cbn_ref[...]) - 2.0 * s
    minv = jnp.min(d, axis=1, keepdims=True)
    ks = lax.broadcasted_iota(jnp.int32, d.shape, 1)
    idx = jnp.min(jnp.where(d == minv, ks, _K), axis=1)
    idx_ref[...] = idx.astype(jnp.int32).reshape(_BM, 1)
    minv_ref[...] = minv.reshape(_BM, 1)


def _encode_vq(patches, pw, pb, maskf, cb, cbn):
    nimg = patches.shape[0]
    ntok = nimg * _H * _H
    return pl.pallas_call(
        _vq_block,
        grid=(nimg // _BI,),
        in_specs=[
            pl.BlockSpec((_BI, 3, _IMG, _IMG), lambda i: (i, 0, 0, 0)),
            pl.BlockSpec((_D, _D), lambda i: (0, 0)),
            pl.BlockSpec((1, _D), lambda i: (0, 0)),
            pl.BlockSpec((_BM, 1), lambda i: (i, 0)),
            pl.BlockSpec((_K, _D), lambda i: (0, 0)),
            pl.BlockSpec((1, _K), lambda i: (0, 0)),
        ],
        out_specs=[
            pl.BlockSpec((_BM, 1), lambda i: (i, 0)),
            pl.BlockSpec((_BM, 1), lambda i: (i, 0)),
        ],
        out_shape=[
            jax.ShapeDtypeStruct((ntok, 1), jnp.int32),
            jax.ShapeDtypeStruct((ntok, 1), jnp.float32),
        ],
    )(patches, pw, pb, maskf, cb, cbn)


def _dec_cb_block(cb_ref, dw_ref, db_ref, out_ref):
    out_ref[...] = lax.dot_general(
        cb_ref[...], dw_ref[...], (((1,), (1,)), ((), ())),
        preferred_element_type=jnp.float32) + db_ref[...]


def _decode_codebook(cb, dw, db):
    return pl.pallas_call(
        _dec_cb_block,
        out_shape=jax.ShapeDtypeStruct((_K, _D), jnp.float32),
    )(cb, dw, db)


def _sc_gather_rows_half(table, idx2):
    mesh = plsc.VectorSubcoreMesh(core_axis_name="c", subcore_axis_name="s")

    @functools.partial(
        pl.kernel,
        out_type=jax.ShapeDtypeStruct((_NW, 224, _D), jnp.float32),
        mesh=mesh,
        scratch_types=[
            pltpu.VMEM((200,), jnp.int32),
            pltpu.VMEM((56, _D), jnp.float32),
            pltpu.SemaphoreType.DMA,
        ],
    )
    def gk(table_hbm, idx_hbm, out_hbm, idx_v, rows_v, sem):
        wid = lax.axis_index("s") * _NC + lax.axis_index("c")
        pltpu.sync_copy(idx_hbm.at[wid], idx_v)
        for j, (off, sz) in enumerate(((0, 56), (56, 56), (112, 56), (168, 32))):
            src = table_hbm.at[idx_v.at[pl.ds(off, sz)]]
            dst = rows_v.at[pl.ds(0, sz)]
            pltpu.async_copy(src, dst, sem).wait()
            pltpu.sync_copy(dst, out_hbm.at[wid, pl.ds(off, sz)])

    return gk(table, idx2)


def kernel(x, patch_w, patch_b, codebook_w, dec_w, dec_b):
    pw = patch_w.reshape(_D, _D)
    pb = patch_b.reshape(1, _D)
    mask = jax.random.uniform(jax.random.key(42), (_B, _H * _H)) < _MASK_RATIO
    maskf = mask.astype(jnp.float32).reshape(_N, 1)
    cbn = jnp.sum(codebook_w ** 2, axis=1).reshape(1, _K)
    dec_cb = _decode_codebook(codebook_w, dec_w, dec_b.reshape(1, _D))

    hb = _B // 2
    ht = _N // 2
    recons, msums = [], []
    for h in range(2):
        idx_h, minv_h = _encode_vq(
            x[h * hb:(h + 1) * hb], pw, pb,
            maskf[h * ht:(h + 1) * ht], codebook_w, cbn)
        idx_p = jnp.pad(idx_h.reshape(_NW, 196), ((0, 0), (0, 4)))
        slab = _sc_gather_rows_half(dec_cb, idx_p)
        toks = slab[:, :196, :].reshape(hb, _H, _H, _P, _P, 3)
        recons.append(toks.transpose(0, 5, 1, 3, 2, 4)
                          .reshape(hb, 3, _IMG, _IMG))
        msums.append(jnp.sum(minv_h))
    recon = jnp.concatenate(recons, axis=0)
    m = (msums[0] + msums[1]) * (1.0 / (_N * _D))
    vq_loss = m + 0.25 * m
    return recon, vq_loss
